# baseline (speedup 1.0000x reference)
import jax
import jax.numpy as jnp
from jax.experimental import pallas as pl
from jax.experimental.pallas import tpu as pltpu

B, T = 4, 4
S, H = 4096, 768
P = 128
NUM_TRAITS = 12
HH = H // 2
CHUNK = 4096
N = B * S // CHUNK
NBUF = 2
SPLIT = 4


def _fused_kernel(idx_ref, table_ref, wp_ref, bp_ref, w1_ref, b1_ref,
                  w2_ref, b2_ref, hs_hbm, out_hbm,
                  in_buf, out_buf, gates_ref, in_sems, out_sems):
    SUB = CHUNK // SPLIT

    def in_copies(i):
        slot = i % NBUF
        return [pltpu.make_async_copy(
            hs_hbm.at[pl.ds(i * CHUNK + k * SUB, SUB), :],
            in_buf.at[slot, pl.ds(k * SUB, SUB), :],
            in_sems.at[slot, k]) for k in range(SPLIT)]

    def out_copies(i):
        slot = i % NBUF
        return [pltpu.make_async_copy(
            out_buf.at[slot, pl.ds(k * SUB, SUB), :],
            out_hbm.at[pl.ds(i * CHUNK + k * SUB, SUB), :],
            out_sems.at[slot, k]) for k in range(SPLIT)]

    def start_all(copies):
        for c in copies:
            c.start()

    def wait_all(copies):
        for c in copies:
            c.wait()

    for k in range(min(NBUF, N)):
        start_all(in_copies(k))

    iota_k = jax.lax.broadcasted_iota(jnp.int32, (B, NUM_TRAITS), 1)
    acc = jnp.zeros((B, NUM_TRAITS), jnp.float32)
    for t in range(T):
        acc = acc + (idx_ref[:, t][:, None] == iota_k).astype(jnp.float32)
    pooled = acc * (1.0 / T)
    pv = jnp.dot(pooled, table_ref[...],
                 preferred_element_type=jnp.float32)
    h = jnp.dot(pv, wp_ref[...],
                preferred_element_type=jnp.float32) + bp_ref[...]
    g = jnp.tanh(jnp.dot(h, w1_ref[...],
                         preferred_element_type=jnp.float32) + b1_ref[...])
    gates_ref[...] = jax.nn.sigmoid(
        jnp.dot(g, w2_ref[...],
                preferred_element_type=jnp.float32) + b2_ref[...])

    for i in range(N):
        slot = i % NBUF
        wait_all(in_copies(i))
        if i >= NBUF:
            wait_all(out_copies(i - NBUF))
        b = (i * CHUNK) // S
        out_buf[slot] = in_buf[slot] * gates_ref[b:b + 1, :]
        start_all(out_copies(i))
        if i + NBUF < N:
            start_all(in_copies(i + NBUF))

    for j in range(max(N - NBUF, 0), N):
        wait_all(out_copies(j))


def kernel(trait_indices, hidden_states, trait_table, W_proj, b_proj,
           W1, b1, W2, b2):
    whole = lambda *_: (0, 0)
    hs2d = hidden_states.reshape(B * S, H)
    out2d = pl.pallas_call(
        _fused_kernel,
        in_specs=[
            pl.BlockSpec((B, T), whole),
            pl.BlockSpec((NUM_TRAITS, P), whole),
            pl.BlockSpec((P, H), whole),
            pl.BlockSpec((1, H), whole),
            pl.BlockSpec((H, HH), whole),
            pl.BlockSpec((1, HH), whole),
            pl.BlockSpec((HH, H), whole),
            pl.BlockSpec((1, H), whole),
            pl.BlockSpec(memory_space=pltpu.MemorySpace.HBM),
        ],
        out_specs=pl.BlockSpec(memory_space=pltpu.MemorySpace.HBM),
        out_shape=jax.ShapeDtypeStruct((B * S, H), jnp.float32),
        scratch_shapes=[
            pltpu.VMEM((NBUF, CHUNK, H), jnp.float32),
            pltpu.VMEM((NBUF, CHUNK, H), jnp.float32),
            pltpu.VMEM((B, H), jnp.float32),
            pltpu.SemaphoreType.DMA((NBUF, SPLIT)),
            pltpu.SemaphoreType.DMA((NBUF, SPLIT)),
        ],
    )(
        trait_indices.astype(jnp.int32),
        trait_table,
        W_proj,
        b_proj.reshape(1, H),
        W1,
        b1.reshape(1, HH),
        W2,
        b2.reshape(1, H),
        hs2d,
    )
    return out2d.reshape(B, S, H)
